# baseline (speedup 1.0000x reference)
import functools

import jax
import jax.numpy as jnp
import numpy as np
from jax import lax
from jax.experimental import pallas as pl
from jax.experimental.pallas import tpu as pltpu
from jax.experimental.pallas import tpu_sc as plsc

N = 10000
E = 320000
NC = 2
NS = 16
NW = NC * NS
CL = 128
CH_TOT = 80
CH0 = 104
CH1 = 56
CH_MAX = max(CH0, CH1)
NCHK = NS * (CH0 + CH1)
NROWS = NCHK + CH_MAX
E_PAD = NROWS * CL
NP = 10112
RPT = NP // NS

_mesh = plsc.VectorSubcoreMesh(core_axis_name="c", subcore_axis_name="s")
_sc_params = pltpu.CompilerParams(use_tc_tiling_on_sc=False,
                                  needs_layout_passes=False,
                                  skip_device_barrier=True,
                                  disable_bounds_checks=True,
                                  disable_semaphore_checks=True)


def _make_deg_kernel():
    @functools.partial(
        pl.kernel,
        out_type=jax.ShapeDtypeStruct((NC * NP,), jnp.float32),
        mesh=_mesh,
        compiler_params=_sc_params,
        scratch_types=[
            pltpu.VMEM((CH_MAX, CL), jnp.int32),
            pltpu.VMEM((CH_MAX, CL), jnp.float32),
            pltpu.VMEM((RPT,), jnp.float32),
            pltpu.VMEM_SHARED((NP,), jnp.float32),
            pltpu.SemaphoreType.DMA,
        ],
    )
    def deg_kernel(c_hbm, w_hbm, z_hbm, out_hbm, c_v, w_v, obuf, acc_sh, sem):
        cid = lax.axis_index("c")
        sid = lax.axis_index("s")
        cbase = jnp.where(cid == 0, sid * CH0, NS * CH0 + sid * CH1)
        mych = jnp.where(cid == 0, CH0, CH1)
        base = pl.multiple_of(sid * RPT, 8)
        pltpu.sync_copy(z_hbm, obuf)
        pltpu.sync_copy(obuf, acc_sh.at[pl.ds(base, RPT)])
        pltpu.sync_copy(c_hbm.at[pl.ds(cbase, CH_MAX)], c_v)
        pltpu.sync_copy(w_hbm.at[pl.ds(cbase, CH_MAX)], w_v)
        plsc.subcore_barrier()

        @pl.loop(0, mych)
        def _fire(j):
            pltpu.async_copy(w_v.at[j], acc_sh.at[c_v.at[j]], sem, add=True)

        @pl.loop(0, mych)
        def _drain(j):
            pltpu.make_async_copy(w_v.at[j], acc_sh.at[c_v.at[j]], sem).wait()

        plsc.subcore_barrier()
        obase = pl.multiple_of(cid * NP + sid * RPT, 8)
        pltpu.sync_copy(acc_sh.at[pl.ds(base, RPT)], obuf)
        pltpu.sync_copy(obuf, out_hbm.at[pl.ds(obase, RPT)])

    return deg_kernel


NB = 4
assert CH0 % NB == 0 and CH1 % NB == 0
ZBLK = NP // CL
BPT = -(-ZBLK // NS)


def _make_msg_kernel(D, bf16):
    in_dtype = jnp.bfloat16 if bf16 else jnp.float32
    scratch = [
        pltpu.VMEM((CH_MAX, CL), jnp.int32),
        pltpu.VMEM((CH_MAX, CL), jnp.int32),
        pltpu.VMEM((CH_MAX, CL), jnp.float32),
        pltpu.VMEM((NB, CL, D), in_dtype),
        pltpu.VMEM((NB, CL, D), jnp.float32),
        pltpu.VMEM_SHARED((NP, D), jnp.float32),
    ]

    @functools.partial(
        pl.kernel,
        out_type=jax.ShapeDtypeStruct((NC, NP, D), jnp.float32),
        mesh=_mesh,
        compiler_params=_sc_params,
        scratch_types=scratch + [pltpu.SemaphoreType.DMA] * (2 * NB),
    )
    def msg_kernel(r_hbm, c_hbm, w_hbm, y_hbm, out_hbm,
                   r_v, c_v, w_v, rows_g, rows_f, acc_sh, *sems):
        gsem = sems[:NB]
        ssem = sems[NB:]
        cid = lax.axis_index("c")
        sid = lax.axis_index("s")
        cbase = jnp.where(cid == 0, sid * CH0, NS * CH0 + sid * CH1)
        myngrp = jnp.where(cid == 0, CH0 // NB, CH1 // NB)

        @pl.loop(0, CL)
        def _zero(rr):
            for q in range(D // 16):
                rows_f[0, rr, pl.ds(q * 16, 16)] = jnp.zeros((16,), jnp.float32)

        for i in range(BPT):
            k = sid * BPT + i

            @pl.when(k < ZBLK)
            def _init():
                pltpu.sync_copy(rows_f.at[0], acc_sh.at[pl.ds(k * CL, CL)])

        pltpu.sync_copy(r_hbm.at[pl.ds(cbase, CH_MAX)], r_v)
        pltpu.sync_copy(c_hbm.at[pl.ds(cbase, CH_MAX)], c_v)
        pltpu.sync_copy(w_hbm.at[pl.ds(cbase, CH_MAX)], w_v)
        plsc.subcore_barrier()

        for b in range(NB):
            pltpu.async_copy(y_hbm.at[r_v.at[b]], rows_g.at[b], gsem[b])

        @pl.loop(0, myngrp)
        def _grp(g):
            j0 = g * NB
            for b in range(NB):
                j = j0 + b
                pltpu.make_async_copy(
                    y_hbm.at[r_v.at[j]], rows_g.at[b], gsem[b]).wait()

                @pl.when(g > 0)
                def _wait_prev_scatter():
                    pltpu.make_async_copy(
                        rows_f.at[b], acc_sh.at[c_v.at[j]], ssem[b]).wait()

                for e0 in range(0, CL, 16):
                    wv = w_v[j, pl.ds(e0, 16)]
                    for t in range(16):
                        ws = wv[t]
                        e = e0 + t
                        if bf16:
                            for q in range(D // 32):
                                blk = rows_g[b, e, pl.ds(q * 32, 32)]
                                lo, hi = plsc.unpack(
                                    blk, format=plsc.PackFormat.INTERLEAVED)
                                rows_f[b, e, pl.ds(q * 32, 16)] = lo * ws
                                rows_f[b, e, pl.ds(q * 32 + 16, 16)] = hi * ws
                        else:
                            for q in range(D // 16):
                                sl = pl.ds(q * 16, 16)
                                rows_f[b, e, sl] = rows_g[b, e, sl] * ws
                pltpu.async_copy(rows_f.at[b], acc_sh.at[c_v.at[j]],
                                 ssem[b], add=True)

                @pl.when(g + 1 < myngrp)
                def _prefetch():
                    pltpu.async_copy(y_hbm.at[r_v.at[j + NB]],
                                     rows_g.at[b], gsem[b])

        for b in range(NB):
            j = (myngrp - 1) * NB + b
            pltpu.make_async_copy(
                rows_f.at[b], acc_sh.at[c_v.at[j]], ssem[b]).wait()

        plsc.subcore_barrier()
        for i in range(BPT):
            k = sid * BPT + i

            @pl.when(k < ZBLK)
            def _flush():
                pltpu.sync_copy(acc_sh.at[pl.ds(k * CL, CL)], rows_f.at[0])
                pltpu.sync_copy(rows_f.at[0], out_hbm.at[cid, pl.ds(k * CL, CL)])

    return msg_kernel


_deg_call = _make_deg_kernel()
_msg_call = {64: _make_msg_kernel(64, True),
             32: _make_msg_kernel(32, True),
             16: _make_msg_kernel(16, False)}


def _ilv(D):
    idx = np.empty((D,), np.int32)
    for q in range(D // 32):
        base = 32 * q
        idx[base:base + 32:2] = np.arange(base, base + 16)
        idx[base + 1:base + 32:2] = np.arange(base + 16, base + 32)
    return idx


_ILV = {64: _ilv(64), 32: _ilv(32)}


def _y1_body(x_ref, w_ref, wp_ref, degt_ref, y_ref, ybf_ref, dis_ref):
    x = x_ref[...]
    dt = degt_ref[...]
    s = dt[:, 0:1] + dt[:, 1:2] + 1.0
    dis = jnp.where(s > 0, lax.rsqrt(s), 0.0)
    dis_ref[...] = dis
    xw = jnp.dot(x, w_ref[...], preferred_element_type=jnp.float32)
    y_ref[...] = dis * xw
    xwp = jnp.dot(x, wp_ref[...], preferred_element_type=jnp.float32)
    ybf_ref[...] = (dis * xwp).astype(jnp.bfloat16)


def _y1_call(x, W1, W1p, degt):
    return pl.pallas_call(
        _y1_body,
        out_shape=[jax.ShapeDtypeStruct((N, W1.shape[1]), jnp.float32),
                   jax.ShapeDtypeStruct((N, W1.shape[1]), jnp.bfloat16),
                   jax.ShapeDtypeStruct((N, 1), jnp.float32)],
    )(x, W1, W1p, degt)


def _mid_body(p0_ref, p1_ref, y_ref, dis_ref, b_ref, w_ref, wp_ref,
              yo_ref, yobf_ref):
    dis = dis_ref[...]
    h = dis * (p0_ref[...] + p1_ref[...] + y_ref[...]) + b_ref[...]
    a = jnp.where(h > 0, h, 0.01 * h)
    yo_ref[...] = dis * jnp.dot(
        a, w_ref[...], preferred_element_type=jnp.float32)
    if wp_ref is not None:
        yobf_ref[...] = (dis * jnp.dot(
            a, wp_ref[...], preferred_element_type=jnp.float32)
        ).astype(jnp.bfloat16)


def _mid_call(p0, p1, y, dis_col, b, W, Wp=None):
    if Wp is None:
        def body2(p0r, p1r, yr, dr, br, wr, yor):
            _mid_body(p0r, p1r, yr, dr, br, wr, None, yor, None)

        return pl.pallas_call(
            body2,
            out_shape=jax.ShapeDtypeStruct((N, W.shape[1]), jnp.float32),
        )(p0, p1, y, dis_col, b, W)
    return pl.pallas_call(
        _mid_body,
        out_shape=[jax.ShapeDtypeStruct((N, W.shape[1]), jnp.float32),
                   jax.ShapeDtypeStruct((N, W.shape[1]), jnp.bfloat16)],
    )(p0, p1, y, dis_col, b, W, Wp)


def _out_body(p0_ref, p1_ref, y_ref, dis_ref, b_ref, o_ref):
    t = dis_ref[...] * (p0_ref[...] + p1_ref[...] + y_ref[...])
    h = t[:, :4] + b_ref[...]
    m = jnp.max(h, axis=1, keepdims=True)
    e = jnp.exp(h - m)
    o_ref[...] = e / jnp.sum(e, axis=1, keepdims=True)


def _out_call(p0, p1, y, dis_col, b3):
    return pl.pallas_call(
        _out_body,
        out_shape=jax.ShapeDtypeStruct((N, 4), jnp.float32),
    )(p0, p1, y, dis_col, b3)


def kernel(x_embeddings, edge_index, weights, W1, b1, W2, b2, W3, b3):
    row = edge_index[0]
    col = edge_index[1]
    pad = E_PAD - E
    rp = jnp.concatenate([row, jnp.zeros((pad,), row.dtype)])
    cp = jnp.concatenate([col, jnp.zeros((pad,), col.dtype)])
    wp = jnp.concatenate([weights, jnp.zeros((pad,), weights.dtype)])
    rp = rp.reshape(NROWS, CL)
    cp = cp.reshape(NROWS, CL)
    wp = wp.reshape(NROWS, CL)

    z1 = jnp.zeros((RPT,), jnp.float32)

    degp = _deg_call(cp, wp, z1).reshape(NC, NP)
    degt = jnp.transpose(degp[:, :N])

    y1, y1bf, dis_col = _y1_call(x_embeddings, W1, W1[:, _ILV[64]], degt)
    p1 = _msg_call[64](rp, cp, wp, y1bf)
    y2, y2bf = _mid_call(p1[0, :N], p1[1, :N], y1, dis_col, b1,
                         W2, W2[:, _ILV[32]])
    p2 = _msg_call[32](rp, cp, wp, y2bf)
    W3p = jnp.pad(W3, ((0, 0), (0, 12)))
    y3 = _mid_call(p2[0, :N], p2[1, :N], y2, dis_col, b2, W3p)
    p3 = _msg_call[16](rp, cp, wp, y3)
    out = _out_call(p3[0, :N], p3[1, :N], y3, dis_col, b3)
    return out


# speedup vs baseline: 1.0752x; 1.0752x over previous
import functools

import jax
import jax.numpy as jnp
import numpy as np
from jax import lax
from jax.experimental import pallas as pl
from jax.experimental.pallas import tpu as pltpu
from jax.experimental.pallas import tpu_sc as plsc

N = 10000
E = 320000
NC = 2
NS = 16
NW = NC * NS
CL = 128
CH_TOT = 80
CH0 = 104
CH1 = 56
CH_MAX = max(CH0, CH1)
NCHK = NS * (CH0 + CH1)
E_PAD = NCHK * CL
NP = 10112
RPT = NP // NS

_mesh = plsc.VectorSubcoreMesh(core_axis_name="c", subcore_axis_name="s")
_sc_params = pltpu.CompilerParams(use_tc_tiling_on_sc=False,
                                  needs_layout_passes=False,
                                  skip_device_barrier=True,
                                  disable_bounds_checks=True,
                                  disable_semaphore_checks=True)


def _make_deg_kernel():
    @functools.partial(
        pl.kernel,
        out_type=jax.ShapeDtypeStruct((NC * NP,), jnp.float32),
        mesh=_mesh,
        compiler_params=_sc_params,
        scratch_types=[
            pltpu.VMEM((CH_MAX, CL), jnp.int32),
            pltpu.VMEM((CH_MAX, CL), jnp.float32),
            pltpu.VMEM((RPT,), jnp.float32),
            pltpu.VMEM_SHARED((NP,), jnp.float32),
            pltpu.SemaphoreType.DMA,
        ],
    )
    def deg_kernel(c_hbm, w_hbm, z_hbm, out_hbm, c_v, w_v, obuf, acc_sh, sem):
        cid = lax.axis_index("c")
        sid = lax.axis_index("s")
        cbase = jnp.where(cid == 0, sid * CH0, NS * CH0 + sid * CH1)
        mych = jnp.where(cid == 0, CH0, CH1)
        sbase = jnp.minimum(cbase, NCHK - CH_MAX)
        off = cbase - sbase
        base = pl.multiple_of(sid * RPT, 8)
        pltpu.sync_copy(z_hbm, obuf)
        pltpu.sync_copy(obuf, acc_sh.at[pl.ds(base, RPT)])
        pltpu.sync_copy(c_hbm.at[pl.ds(sbase, CH_MAX)], c_v)
        pltpu.sync_copy(w_hbm.at[pl.ds(sbase, CH_MAX)], w_v)
        plsc.subcore_barrier()

        @pl.loop(0, mych)
        def _fire(j):
            pltpu.async_copy(w_v.at[off + j], acc_sh.at[c_v.at[off + j]],
                             sem, add=True)

        @pl.loop(0, mych)
        def _drain(j):
            pltpu.make_async_copy(w_v.at[off + j],
                                  acc_sh.at[c_v.at[off + j]], sem).wait()

        plsc.subcore_barrier()
        obase = pl.multiple_of(cid * NP + sid * RPT, 8)
        pltpu.sync_copy(acc_sh.at[pl.ds(base, RPT)], obuf)
        pltpu.sync_copy(obuf, out_hbm.at[pl.ds(obase, RPT)])

    return deg_kernel


NB = 4
assert CH0 % NB == 0 and CH1 % NB == 0
ZBLK = NP // CL
BPT = -(-ZBLK // NS)


def _make_msg_kernel(D, bf16):
    in_dtype = jnp.bfloat16 if bf16 else jnp.float32
    scratch = [
        pltpu.VMEM((CH_MAX, CL), jnp.int32),
        pltpu.VMEM((CH_MAX, CL), jnp.int32),
        pltpu.VMEM((CH_MAX, CL), jnp.float32),
        pltpu.VMEM((NB, CL, D), in_dtype),
        pltpu.VMEM((NB, CL, D), jnp.float32),
        pltpu.VMEM_SHARED((NP, D), jnp.float32),
    ]

    @functools.partial(
        pl.kernel,
        out_type=jax.ShapeDtypeStruct((NC, NP, D), jnp.float32),
        mesh=_mesh,
        compiler_params=_sc_params,
        scratch_types=scratch + [pltpu.SemaphoreType.DMA] * (2 * NB),
    )
    def msg_kernel(r_hbm, c_hbm, w_hbm, y_hbm, out_hbm,
                   r_v, c_v, w_v, rows_g, rows_f, acc_sh, *sems):
        gsem = sems[:NB]
        ssem = sems[NB:]
        cid = lax.axis_index("c")
        sid = lax.axis_index("s")
        cbase = jnp.where(cid == 0, sid * CH0, NS * CH0 + sid * CH1)
        myngrp = jnp.where(cid == 0, CH0 // NB, CH1 // NB)
        sbase = jnp.minimum(cbase, NCHK - CH_MAX)
        off = cbase - sbase

        @pl.loop(0, CL)
        def _zero(rr):
            for q in range(D // 16):
                rows_f[0, rr, pl.ds(q * 16, 16)] = jnp.zeros((16,), jnp.float32)

        for i in range(BPT):
            k = sid * BPT + i

            @pl.when(k < ZBLK)
            def _init():
                pltpu.sync_copy(rows_f.at[0], acc_sh.at[pl.ds(k * CL, CL)])

        pltpu.sync_copy(r_hbm.at[pl.ds(sbase, CH_MAX)], r_v)
        pltpu.sync_copy(c_hbm.at[pl.ds(sbase, CH_MAX)], c_v)
        pltpu.sync_copy(w_hbm.at[pl.ds(sbase, CH_MAX)], w_v)
        plsc.subcore_barrier()

        for b in range(NB):
            pltpu.async_copy(y_hbm.at[r_v.at[off + b]], rows_g.at[b], gsem[b])

        @pl.loop(0, myngrp)
        def _grp(g):
            j0 = g * NB
            for b in range(NB):
                j = off + j0 + b
                pltpu.make_async_copy(
                    y_hbm.at[r_v.at[j]], rows_g.at[b], gsem[b]).wait()

                @pl.when(g > 0)
                def _wait_prev_scatter():
                    pltpu.make_async_copy(
                        rows_f.at[b], acc_sh.at[c_v.at[j]], ssem[b]).wait()

                for e0 in range(0, CL, 16):
                    wv = w_v[j, pl.ds(e0, 16)]
                    for t in range(16):
                        ws = wv[t]
                        e = e0 + t
                        if bf16:
                            for q in range(D // 32):
                                blk = rows_g[b, e, pl.ds(q * 32, 32)]
                                lo, hi = plsc.unpack(
                                    blk, format=plsc.PackFormat.INTERLEAVED)
                                rows_f[b, e, pl.ds(q * 32, 16)] = lo * ws
                                rows_f[b, e, pl.ds(q * 32 + 16, 16)] = hi * ws
                        else:
                            for q in range(D // 16):
                                sl = pl.ds(q * 16, 16)
                                rows_f[b, e, sl] = rows_g[b, e, sl] * ws
                pltpu.async_copy(rows_f.at[b], acc_sh.at[c_v.at[j]],
                                 ssem[b], add=True)

                @pl.when(g + 1 < myngrp)
                def _prefetch():
                    pltpu.async_copy(y_hbm.at[r_v.at[j + NB]],
                                     rows_g.at[b], gsem[b])

        for b in range(NB):
            j = off + (myngrp - 1) * NB + b
            pltpu.make_async_copy(
                rows_f.at[b], acc_sh.at[c_v.at[j]], ssem[b]).wait()

        plsc.subcore_barrier()
        for i in range(BPT):
            k = sid * BPT + i

            @pl.when(k < ZBLK)
            def _flush():
                pltpu.sync_copy(acc_sh.at[pl.ds(k * CL, CL)], rows_f.at[0])
                pltpu.sync_copy(rows_f.at[0], out_hbm.at[cid, pl.ds(k * CL, CL)])

    return msg_kernel


_deg_call = _make_deg_kernel()
_msg_call = {64: _make_msg_kernel(64, True),
             32: _make_msg_kernel(32, True),
             16: _make_msg_kernel(16, False)}


def _ilv(D):
    idx = np.empty((D,), np.int32)
    for q in range(D // 32):
        base = 32 * q
        idx[base:base + 32:2] = np.arange(base, base + 16)
        idx[base + 1:base + 32:2] = np.arange(base + 16, base + 32)
    return idx


_ILV = {64: _ilv(64), 32: _ilv(32)}


def _y1_body(x_ref, w_ref, wp_ref, degt_ref, y_ref, ybf_ref, dis_ref):
    x = x_ref[...]
    dt = degt_ref[...]
    s = dt[:, 0:1] + dt[:, 1:2] + 1.0
    dis = jnp.where(s > 0, lax.rsqrt(s), 0.0)
    dis_ref[...] = dis
    xw = jnp.dot(x, w_ref[...], preferred_element_type=jnp.float32)
    y_ref[...] = dis * xw
    xwp = jnp.dot(x, wp_ref[...], preferred_element_type=jnp.float32)
    ybf_ref[...] = (dis * xwp).astype(jnp.bfloat16)


def _y1_call(x, W1, W1p, degt):
    return pl.pallas_call(
        _y1_body,
        out_shape=[jax.ShapeDtypeStruct((N, W1.shape[1]), jnp.float32),
                   jax.ShapeDtypeStruct((N, W1.shape[1]), jnp.bfloat16),
                   jax.ShapeDtypeStruct((N, 1), jnp.float32)],
    )(x, W1, W1p, degt)


def _mid_body(p0_ref, p1_ref, y_ref, dis_ref, b_ref, w_ref, wp_ref,
              yo_ref, yobf_ref):
    dis = dis_ref[...]
    h = dis * (p0_ref[...] + p1_ref[...] + y_ref[...]) + b_ref[...]
    a = jnp.where(h > 0, h, 0.01 * h)
    yo_ref[...] = dis * jnp.dot(
        a, w_ref[...], preferred_element_type=jnp.float32)
    if wp_ref is not None:
        yobf_ref[...] = (dis * jnp.dot(
            a, wp_ref[...], preferred_element_type=jnp.float32)
        ).astype(jnp.bfloat16)


def _mid_call(p0, p1, y, dis_col, b, W, Wp=None):
    if Wp is None:
        def body2(p0r, p1r, yr, dr, br, wr, yor):
            _mid_body(p0r, p1r, yr, dr, br, wr, None, yor, None)

        return pl.pallas_call(
            body2,
            out_shape=jax.ShapeDtypeStruct((N, W.shape[1]), jnp.float32),
        )(p0, p1, y, dis_col, b, W)
    return pl.pallas_call(
        _mid_body,
        out_shape=[jax.ShapeDtypeStruct((N, W.shape[1]), jnp.float32),
                   jax.ShapeDtypeStruct((N, W.shape[1]), jnp.bfloat16)],
    )(p0, p1, y, dis_col, b, W, Wp)


def _out_body(p0_ref, p1_ref, y_ref, dis_ref, b_ref, o_ref):
    t = dis_ref[...] * (p0_ref[...] + p1_ref[...] + y_ref[...])
    h = t[:, :4] + b_ref[...]
    m = jnp.max(h, axis=1, keepdims=True)
    e = jnp.exp(h - m)
    o_ref[...] = e / jnp.sum(e, axis=1, keepdims=True)


def _out_call(p0, p1, y, dis_col, b3):
    return pl.pallas_call(
        _out_body,
        out_shape=jax.ShapeDtypeStruct((N, 4), jnp.float32),
    )(p0, p1, y, dis_col, b3)


def kernel(x_embeddings, edge_index, weights, W1, b1, W2, b2, W3, b3):
    row = edge_index[0]
    col = edge_index[1]
    pad = E_PAD - E
    rp = jnp.concatenate([row, jnp.zeros((pad,), row.dtype)])
    cp = jnp.concatenate([col, jnp.zeros((pad,), col.dtype)])
    wp = jnp.concatenate([weights, jnp.zeros((pad,), weights.dtype)])
    rp = rp.reshape(NCHK, CL)
    cp = cp.reshape(NCHK, CL)
    wp = wp.reshape(NCHK, CL)

    z1 = jnp.zeros((RPT,), jnp.float32)

    degp = _deg_call(cp, wp, z1).reshape(NC, NP)
    degt = jnp.transpose(degp[:, :N])

    y1, y1bf, dis_col = _y1_call(x_embeddings, W1, W1[:, _ILV[64]], degt)
    p1 = _msg_call[64](rp, cp, wp, y1bf)
    y2, y2bf = _mid_call(p1[0, :N], p1[1, :N], y1, dis_col, b1,
                         W2, W2[:, _ILV[32]])
    p2 = _msg_call[32](rp, cp, wp, y2bf)
    W3p = jnp.pad(W3, ((0, 0), (0, 12)))
    y3 = _mid_call(p2[0, :N], p2[1, :N], y2, dis_col, b2, W3p)
    p3 = _msg_call[16](rp, cp, wp, y3)
    out = _out_call(p3[0, :N], p3[1, :N], y3, dis_col, b3)
    return out
